# Initial kernel scaffold; baseline (speedup 1.0000x reference)
import jax
import jax.numpy as jnp
from jax.experimental import pallas as pl


def kernel(image):
    raise NotImplementedError("write your pallas kernel here")


# speedup vs baseline: 1.0068x; 1.0068x over previous
import jax
import jax.numpy as jnp
import numpy as np
from jax.experimental import pallas as pl

_TILE = 64
_OVERLAP = 16
_RADII = [1, 2, 3, 4, 5, 6]
_RW = [1.0, 0.6, 0.4, 0.3, 0.2, 0.1]
_EDGE_T = 0.15
_M = 20
_TOL = 1e-10
_HW = 224
_N = _TILE * _TILE


def _build_offsets():
    dys, dxs, ws = [], [], []
    for r, rw in zip(_RADII, _RW):
        for dy in range(-r, r + 1):
            for dx in range(-r, r + 1):
                if (dx == 0 and dy == 0) or dx * dx + dy * dy > r * r:
                    continue
                dys.append(dy); dxs.append(dx); ws.append(rw)
    return np.array(dys, np.int64), np.array(dxs, np.int64), np.array(ws, np.float32)


_OFF_DY, _OFF_DX, _OFF_W = _build_offsets()
_K = _OFF_DY.shape[0]


def _build_ell():
    H = W = _TILE
    n = H * W
    yy = np.repeat(np.arange(H), W)
    xx = np.tile(np.arange(W), H)
    dy = yy[:, None] + _OFF_DY[None, :]
    dx = xx[:, None] + _OFF_DX[None, :]
    valid = (dy >= 0) & (dy < H) & (dx >= 0) & (dx < W)
    col = np.clip(dy * W + dx, 0, n - 1).astype(np.int32)
    base_w = np.broadcast_to(_OFF_W[None, :], (n, _OFF_W.shape[0])).astype(np.float32)
    return col, valid.astype(np.float32), base_w


_ELL_COL, _ELL_VALID, _ELL_BASE_W = _build_ell()


def _positions(L):
    stride = _TILE - _OVERLAP
    ps = list(range(0, L - _TILE + 1, stride))
    if ps[-1] != L - _TILE:
        ps.append(L - _TILE)
    return ps


def _wmat_body(f_ref, ng_ref, valid_ref, bw_ref, o_ref):
    f = f_ref[0, 0]
    ng = ng_ref[0]
    o_ref[0] = bw_ref[...] * jnp.exp(-jnp.abs(f[:, None] - ng) / _EDGE_T) * valid_ref[...]


def _wmat_pallas(flatB, neighB):
    B = flatB.shape[0]
    flatB = flatB[:, None, :]
    return pl.pallas_call(
        _wmat_body,
        grid=(B,),
        in_specs=[
            pl.BlockSpec((1, 1, _N), lambda i: (i, 0, 0)),
            pl.BlockSpec((1, _N, _K), lambda i: (i, 0, 0)),
            pl.BlockSpec((_N, _K), lambda i: (0, 0)),
            pl.BlockSpec((_N, _K), lambda i: (0, 0)),
        ],
        out_specs=pl.BlockSpec((1, _N, _K), lambda i: (i, 0, 0)),
        out_shape=jax.ShapeDtypeStruct((B, _N, _K), jnp.float32),
    )(flatB, neighB, jnp.asarray(_ELL_VALID), jnp.asarray(_ELL_BASE_W))


def _tile_fiedler_batched(tiles, v0):
    n = _N
    col = jnp.asarray(_ELL_COL)
    flatB = tiles.reshape(tiles.shape[0], n)
    neighB = jax.vmap(lambda f: f[col])(flatB)
    wmatB = _wmat_pallas(flatB, neighB)
    degB = jax.vmap(lambda w: w.sum(axis=1))(wmatB)
    ones_n = jnp.full((n,), 1.0 / np.sqrt(n), dtype=jnp.float32)

    def tile_lanczos(flat, wmat, deg):
        def matvec(v):
            return deg * v - (wmat * v[col]).sum(axis=1)

        v = v0 - jnp.dot(v0, ones_n) * ones_n
        v = v / jnp.maximum(jnp.linalg.norm(v), _TOL)
        V0 = jnp.zeros((_M, n), dtype=flat.dtype)

        def body(carry, i):
            v_prev, v_cur, beta_prev, V = carry
            V = V.at[i].set(v_cur)
            r = matvec(v_cur)
            a = jnp.dot(r, v_cur)
            r = r - a * v_cur - beta_prev * v_prev
            r = r - jnp.dot(r, ones_n) * ones_n
            r = r - V.T @ (V @ r)
            b = jnp.linalg.norm(r)
            v_next = r / jnp.maximum(b, _TOL)
            return (v_cur, v_next, b, V), (a, b)

        init = (jnp.zeros((n,), flat.dtype), v, jnp.asarray(0.0, flat.dtype), V0)
        (_, _, _, V), (alphas, betas) = jax.lax.scan(jax.checkpoint(body), init, jnp.arange(_M))
        T = jnp.diag(alphas) + jnp.diag(betas[:-1], 1) + jnp.diag(betas[:-1], -1)
        _, evecs = jnp.linalg.eigh(T)
        fied = V.T @ evecs[:, 0]
        s = jnp.where(jnp.sum(fied) >= 0.0, 1.0, -1.0)
        fied = fied * jax.lax.stop_gradient(s)
        return fied.reshape(_TILE, _TILE)

    return jax.vmap(tile_lanczos)(flatB, wmatB, degB)


def kernel(image):
    ys = _positions(_HW)
    xs = _positions(_HW)
    tiles = jnp.stack([image[y:y + _TILE, x:x + _TILE] for y in ys for x in xs])
    rng = np.random.default_rng(0)
    v0 = jnp.asarray(rng.standard_normal(_N).astype(np.float32))
    fied = _tile_fiedler_batched(tiles, v0)
    r = np.arange(_TILE, dtype=np.float32)
    ramp = np.minimum(np.minimum((r + 1.0) / (_OVERLAP + 1.0), (_TILE - r) / (_OVERLAP + 1.0)), 1.0)
    taper = jnp.asarray(np.outer(ramp, ramp).astype(np.float32))
    out = jnp.zeros((_HW, _HW), dtype=image.dtype)
    norm = jnp.zeros((_HW, _HW), dtype=image.dtype)
    k = 0
    for y in ys:
        for x in xs:
            out = out.at[y:y + _TILE, x:x + _TILE].add(fied[k] * taper)
            norm = norm.at[y:y + _TILE, x:x + _TILE].add(taper)
            k += 1
    return out / jnp.maximum(norm, 1e-8)
